# baseline (speedup 1.0000x reference)
import jax
import jax.numpy as jnp
from jax.experimental import pallas as pl
from jax.experimental.pallas import tpu as pltpu

B, L, H, K, D = 64, 1024, 32, 100000, 256
BLK = 4096
NK = (K + BLK - 1) // BLK

_NEG_INF = float("-inf")


def _logits_kernel(x_ref, w1_ref, b1p_ref, w2_ref, b2_ref,
                   logits_ref, lse_ref, actions_ref,
                   h_s, m_s, s_s, am_s):
    k = pl.program_id(0)

    @pl.when(k == 0)
    def _init():
        h = jnp.maximum(
            jnp.dot(x_ref[...], w1_ref[...], preferred_element_type=jnp.float32)
            + b1p_ref[...], 0.0)
        h_s[...] = h
        m_s[...] = jnp.full((B, 1), _NEG_INF, jnp.float32)
        s_s[...] = jnp.zeros((B, 1), jnp.float32)
        am_s[...] = jnp.zeros((B, 1), jnp.int32)

    logits = (jnp.dot(h_s[...], w2_ref[...], preferred_element_type=jnp.float32)
              + b2_ref[...])
    cols = k * BLK + jax.lax.broadcasted_iota(jnp.int32, (B, BLK), 1)
    logits = jax.lax.cond(
        k == NK - 1,
        lambda l: jnp.where(cols < K, l, _NEG_INF),
        lambda l: l,
        logits)
    logits_ref[...] = logits

    bm = jnp.max(logits, axis=1, keepdims=True)
    e = jnp.exp(logits - bm)

    cand = jnp.where(logits == bm, cols, K)
    bam = jnp.min(cand, axis=1, keepdims=True)

    m_old = m_s[...]
    better = bm > m_old
    am_s[...] = jnp.where(better, bam, am_s[...])
    m_new = jnp.maximum(m_old, bm)
    s_s[...] = (s_s[...] * jnp.exp(m_old - m_new)
                + jnp.sum(e, axis=1, keepdims=True) * jnp.exp(bm - m_new))
    m_s[...] = m_new

    @pl.when(k == NK - 1)
    def _final():
        lse_ref[...] = m_s[...] + jnp.log(s_s[...])
        actions_ref[...] = am_s[...].astype(jnp.float32)


def _probs_kernel(logits_ref, lse_ref, probs_ref):
    probs_ref[...] = jnp.exp(logits_ref[...] - lse_ref[...])


def _forward(x, W1, b1p, W2, b2row):
    logits, lse, actions = pl.pallas_call(
        _logits_kernel,
        grid=(NK,),
        in_specs=[
            pl.BlockSpec((B, L + 2 * H), lambda k: (0, 0)),
            pl.BlockSpec((L + 2 * H, D), lambda k: (0, 0)),
            pl.BlockSpec((1, D), lambda k: (0, 0)),
            pl.BlockSpec((D, BLK), lambda k: (0, k)),
            pl.BlockSpec((1, BLK), lambda k: (0, k)),
        ],
        out_specs=[
            pl.BlockSpec((B, BLK), lambda k: (0, k)),
            pl.BlockSpec((B, 1), lambda k: (0, 0)),
            pl.BlockSpec((B, 1), lambda k: (0, 0)),
        ],
        out_shape=[
            jax.ShapeDtypeStruct((B, K), jnp.float32),
            jax.ShapeDtypeStruct((B, 1), jnp.float32),
            jax.ShapeDtypeStruct((B, 1), jnp.float32),
        ],
        scratch_shapes=[
            pltpu.VMEM((B, D), jnp.float32),
            pltpu.VMEM((B, 1), jnp.float32),
            pltpu.VMEM((B, 1), jnp.float32),
            pltpu.VMEM((B, 1), jnp.int32),
        ],
    )(x, W1, b1p, W2, b2row)

    probs = pl.pallas_call(
        _probs_kernel,
        grid=(NK,),
        in_specs=[
            pl.BlockSpec((B, BLK), lambda k: (0, k)),
            pl.BlockSpec((B, 1), lambda k: (0, 0)),
        ],
        out_specs=pl.BlockSpec((B, BLK), lambda k: (0, k)),
        out_shape=jax.ShapeDtypeStruct((B, K), jnp.float32),
    )(logits, lse)
    return logits, probs, actions


def kernel(context, forecast, forecast_mask, step, W1, b1, W2, b2, pos_emb):
    m = forecast_mask.astype(jnp.float32)
    x = jnp.concatenate([context, forecast * m, m], axis=-1)
    b1p = (b1 + pos_emb[step]).reshape(1, D)
    b2row = b2.reshape(1, K)
    logits, probs, actions = _forward(x, W1, b1p, W2, b2row)
    return (actions.reshape(B), probs, logits)
